# Initial kernel scaffold; baseline (speedup 1.0000x reference)
import jax
import jax.numpy as jnp
from jax.experimental import pallas as pl


def kernel(table, x):
    raise NotImplementedError("write your pallas kernel here")


# speedup vs baseline: 3.3370x; 3.3370x over previous
import functools

import jax
import jax.numpy as jnp
from jax import lax
from jax.experimental import pallas as pl
from jax.experimental.pallas import tpu as pltpu
from jax.experimental.pallas import tpu_sc as plsc

D = 128
CH = 128


def _gather_body(n_ch, per_w, nc, table_hbm, idx_hbm, out_hbm,
                 idx_v, buf0, buf1, sem0, sem1):
    wid = lax.axis_index("s") * nc + lax.axis_index("c")
    base = wid * per_w
    pltpu.sync_copy(idx_hbm.at[wid], idx_v)

    def start(g, buf, sem):
        return pltpu.async_copy(table_hbm.at[idx_v.at[g]], buf, sem)

    def drain(g, buf, sem):
        pltpu.make_async_copy(table_hbm.at[idx_v.at[g]], buf, sem).wait()
        pltpu.sync_copy(buf, out_hbm.at[pl.ds(base + g * CH, CH)])

    start(0, buf0, sem0)

    def pair(i, _):
        g0 = 2 * i
        start(g0 + 1, buf1, sem1)
        drain(g0, buf0, sem0)

        @pl.when(g0 + 2 < n_ch)
        def _():
            start(g0 + 2, buf0, sem0)

        drain(g0 + 1, buf1, sem1)
        return 0

    lax.fori_loop(0, n_ch // 2, pair, 0)


def kernel(table, x):
    B, H = x.shape
    N = B * H
    info = plsc.get_sparse_core_info()
    nc, ns = info.num_cores, info.num_subcores
    nw = nc * ns
    per_w = N // nw
    n_ch = per_w // CH
    idx = x.reshape(nw, n_ch, CH).astype(jnp.int32)

    mesh = plsc.VectorSubcoreMesh(core_axis_name="c", subcore_axis_name="s")
    body = functools.partial(_gather_body, n_ch, per_w, nc)
    out = pl.kernel(
        body,
        mesh=mesh,
        out_type=jax.ShapeDtypeStruct((N, D), jnp.float32),
        scratch_types=[
            pltpu.VMEM((n_ch, CH), jnp.int32),
            pltpu.VMEM((CH, D), jnp.float32),
            pltpu.VMEM((CH, D), jnp.float32),
            pltpu.SemaphoreType.DMA,
            pltpu.SemaphoreType.DMA,
        ],
    )(table, idx)
    return out.reshape(B, H, D)
